# baseline (speedup 1.0000x reference)
import functools

import jax
import jax.numpy as jnp
from jax import lax
from jax.experimental import pallas as pl
from jax.experimental.pallas import tpu as pltpu
from jax.experimental.pallas import tpu_sc as plsc

H = 8
C = 64
HALF = C // 2
AW = 16
B = 64
OUT = 10
RELS = [('control', 'control', 'cc'), ('control', 'control', 'call'),
        ('control', 'variable', 'cv'), ('variable', 'control', 'vc'),
        ('constant', 'control', 'kc'), ('control', 'constant', 'ck')]
NTYPES = ('control', 'variable', 'constant')
NSC = 2
NTILES = 16
ZR = 128


def _tc_matmul(x, w, widths, bsum=None, block_r=512):
    n, d = x.shape
    tot = w.shape[1]
    nin = 2 + (1 if bsum is not None else 0)

    def body(*refs):
        xb = refs[0][...]
        if bsum is not None:
            xb = xb + refs[2][...]
            xb = jnp.where(xb > 0, xb, jnp.exp(jnp.minimum(xb, 0.0)) - 1.0)
        acc = jnp.dot(xb, refs[1][...], preferred_element_type=jnp.float32)
        off = 0
        for o_ref in refs[nin:]:
            wd = o_ref.shape[1]
            o_ref[...] = acc[:, off:off + wd]
            off += wd

    in_specs = [pl.BlockSpec((block_r, d), lambda i: (i, 0)),
                pl.BlockSpec((d, tot), lambda i: (0, 0))]
    args = [x, w]
    if bsum is not None:
        in_specs.append(pl.BlockSpec((1, d), lambda i: (0, 0)))
        args.append(bsum)
    return pl.pallas_call(
        body,
        grid=(pl.cdiv(n, block_r),),
        in_specs=in_specs,
        out_specs=[pl.BlockSpec((block_r, wd), lambda i: (i, 0)) for wd in widths],
        out_shape=[jax.ShapeDtypeStruct((n, wd), jnp.float32) for wd in widths],
    )(*args)


def _build_wcat(lp, d_in):
    perm_a = [h * C + c for h in range(H) for c in range(HALF)]
    perm_b = [h * C + HALF + c for h in range(H) for c in range(HALF)]
    wcat, widths = {}, {}
    for t in NTYPES:
        src_rels = [name for (s, _, name) in RELS if s == t]
        dst_rels = [name for (_, dt, name) in RELS if dt == t]
        acols, awd = [], []
        for name in src_rels:
            p = lp[name]
            ws2 = jnp.einsum('dhc,hc->dh', p['w_src'].reshape(d_in, H, C),
                             p['att_src'][0])
            acols.append(jnp.pad(ws2, ((0, 0), (0, AW - H)))); awd.append(AW)
        for name in dst_rels:
            p = lp[name]
            wd2 = jnp.einsum('dhc,hc->dh', p['w_dst'].reshape(d_in, H, C),
                             p['att_dst'][0])
            acols.append(jnp.pad(wd2, ((0, 0), (0, AW - H)))); awd.append(AW)
        hcols, hwd = [], []
        for name in src_rels:
            w = lp[name]['w_src']
            hcols.append(w[:, perm_a]); hwd.append(H * HALF)
            hcols.append(w[:, perm_b]); hwd.append(H * HALF)
        wcat[t] = (jnp.concatenate(acols, axis=1),
                   jnp.concatenate(hcols, axis=1))
        widths[t] = (awd, hwd)
    return wcat, widths


def _tile_rows(nd):
    r8 = -(-(nd // NTILES) // 8) * 8
    last = nd - (NTILES - 1) * r8
    assert last >= 0 and last % 8 == 0
    return r8, last


def _zero_rows(acc, row0, nrows, zbuf):
    zr = zbuf.shape[0]
    nz, rem = nrows // zr, nrows % zr

    def zk(k, _):
        pltpu.sync_copy(zbuf, acc.at[pl.ds(row0 + k * zr, zr)])
        return 0
    lax.fori_loop(0, nz, zk, 0)
    if rem:
        pltpu.sync_copy(zbuf.at[pl.ds(0, rem)],
                        acc.at[pl.ds(row0 + nz * zr, rem)])


def _per_tile_rows(s, acc_nd, fn):
    r8, last = _tile_rows(acc_nd)

    @pl.when(s < NTILES - 1)
    def _():
        fn(s * r8, r8)

    @pl.when(s == NTILES - 1)
    def _():
        fn((NTILES - 1) * r8, last)


def _fill_zeros(zbuf):
    z = jnp.zeros((16,), jnp.float32)

    def zrow(i, _):
        for w in range(zbuf.shape[1] // 16):
            zbuf[i, pl.ds(w * 16, 16)] = z
        return 0
    lax.fori_loop(0, zbuf.shape[0], zrow, 0)


def _sc_pass1(rels):
    nrel = len(rels)
    nr = nrel // NSC
    assert nr * NSC == nrel
    E = rels[0][0].shape[0]
    K = 80
    nfull = E // K
    assert nfull * K == E and K % 8 == 0
    nds = [r[3].shape[0] for r in rels]
    acc_nds = [max(nds[j], nds[nr + j]) for j in range(nr)]

    out_type = []
    for nd in nds:
        out_type += [jax.ShapeDtypeStruct((E, AW), jnp.float32),
                     jax.ShapeDtypeStruct((nd, AW), jnp.float32)]
    p1set = [pltpu.VMEM((K,), jnp.int32), pltpu.VMEM((K,), jnp.int32),
             pltpu.VMEM((K, AW), jnp.float32), pltpu.VMEM((K, AW), jnp.float32)]
    scratch = (p1set + p1set
               + [pltpu.VMEM((K, AW), jnp.float32),
                  pltpu.VMEM((ZR, AW), jnp.float32),
                  pltpu.SemaphoreType.DMA, pltpu.SemaphoreType.DMA])
    scratch += [pltpu.VMEM_SHARED((nd, AW), jnp.float32) for nd in acc_nds]

    def body(*refs):
        ins = refs[:4 * nrel]
        outs = refs[4 * nrel:4 * nrel + 2 * nrel]
        scr = refs[4 * nrel + 2 * nrel:]
        bufs = (scr[0:4], scr[4:8])
        exb, zbuf = scr[8:10]
        sems = (scr[10], scr[11])
        accs = scr[12:]

        c = lax.axis_index('c')
        s = lax.axis_index('s')
        _fill_zeros(zbuf)

        def process(r, acc):
            src_h, dst_h, as_h, ad_h = ins[4 * r:4 * r + 4]
            ex_h = outs[2 * r]

            def start(base, p):
                bi_s, bi_d, ba_s, ba_d = bufs[p]
                sem = sems[p]
                i1 = pltpu.async_copy(src_h.at[pl.ds(base, K)], bi_s, sem)
                i2 = pltpu.async_copy(dst_h.at[pl.ds(base, K)], bi_d, sem)
                i1.wait(); i2.wait()
                pltpu.async_copy(as_h.at[bi_s], ba_s, sem)
                pltpu.async_copy(ad_h.at[bi_d], ba_d, sem)

            def finish(base, p):
                bi_s, bi_d, ba_s, ba_d = bufs[p]
                sem = sems[p]
                pltpu.make_async_copy(as_h.at[bi_s], ba_s, sem).wait()
                pltpu.make_async_copy(ad_h.at[bi_d], ba_d, sem).wait()

                def row(i, _):
                    v = ba_s[i, :] + ba_d[i, :]
                    v = jnp.where(v > 0, v, 0.2 * v)
                    exb[i, :] = jnp.exp(v)
                    return 0
                lax.fori_loop(0, K, row, 0)
                pltpu.sync_copy(exb, acc.at[bi_d], add=True)
                pltpu.sync_copy(exb, ex_h.at[pl.ds(base, K)])

            nb = (nfull - 1 - s) // NTILES + 1

            def base_of(k):
                return (s + k * NTILES) * K

            start(base_of(0), 0)

            def kstep(k, _):
                for p in range(2):
                    @pl.when(k % 2 == p)
                    def _(p=p):
                        @pl.when(k + 1 < nb)
                        def _():
                            start(base_of(k + 1), 1 - p)
                        finish(base_of(k), p)
                return 0
            lax.fori_loop(0, nb, kstep, 0)

        for sc in range(NSC):
            @pl.when(c == sc)
            def _(sc=sc):
                for j in range(nr):
                    r = sc * nr + j
                    _per_tile_rows(s, nds[r],
                                   lambda r0, nrw, j=j: _zero_rows(
                                       accs[j], r0, nrw, zbuf))
        plsc.subcore_barrier()

        for sc in range(NSC):
            @pl.when(c == sc)
            def _(sc=sc):
                for j in range(nr):
                    process(sc * nr + j, accs[j])
        plsc.subcore_barrier()

        for sc in range(NSC):
            @pl.when(c == sc)
            def _(sc=sc):
                for j in range(nr):
                    r = sc * nr + j
                    dn = outs[2 * r + 1]

                    def dump(r0, nrw, j=j, dn=dn):
                        pltpu.sync_copy(accs[j].at[pl.ds(r0, nrw)],
                                        dn.at[pl.ds(r0, nrw)])
                    _per_tile_rows(s, nds[r], dump)

    mesh = plsc.VectorSubcoreMesh(core_axis_name='c', subcore_axis_name='s')
    flat_in = [a for r in rels for a in r]
    outs = pl.kernel(body, out_type=out_type, mesh=mesh,
                     compiler_params=pltpu.CompilerParams(
                         use_tc_tiling_on_sc=False),
                     scratch_types=scratch)(*flat_in)
    return [tuple(outs[2 * r:2 * r + 2]) for r in range(nrel)]


def _sc_pass2(groups, K=64, zr=ZR):
    E = groups[0][1][0][0].shape[0]
    nfull = E // K
    assert nfull * K == E

    out_type = [jax.ShapeDtypeStruct((nd, C), jnp.float32)
                for nd, _ in groups]
    bufset = [pltpu.VMEM((K,), jnp.int32), pltpu.VMEM((K,), jnp.int32),
              pltpu.VMEM((K, H * HALF), jnp.float32),
              pltpu.VMEM((K, AW), jnp.float32),
              pltpu.VMEM((K, AW), jnp.float32)]
    scratch = (bufset + bufset
               + [pltpu.VMEM((K, HALF), jnp.float32),
                  pltpu.VMEM((zr, HALF), jnp.float32),
                  pltpu.SemaphoreType.DMA, pltpu.SemaphoreType.DMA])
    scratch += [pltpu.VMEM_SHARED((nd, HALF), jnp.float32) for nd, _ in groups]

    nin = sum(6 * len(rels) for _, rels in groups)

    def body(*refs):
        ins = refs[:nin]
        outs = refs[nin:nin + len(groups)]
        scr = refs[nin + len(groups):]
        bufs = (scr[0:5], scr[5:10])
        mb, zbuf = scr[10:12]
        sems = (scr[12], scr[13])
        accs = scr[14:]

        c = lax.axis_index('c')
        s = lax.axis_index('s')
        _fill_zeros(zbuf)
        for acc in accs:
            _per_tile_rows(s, acc.shape[0],
                           lambda r0, nr, acc=acc: _zero_rows(acc, r0, nr, zbuf))
        plsc.subcore_barrier()

        off = 0
        for gi, (nd, rels) in enumerate(groups):
            acc = accs[gi]
            for _ in rels:
                src_h, dst_h, hsa_h, hsb_h, ex_h, da_h = ins[off:off + 6]
                off += 6

                def start(base, p, src_h=src_h, dst_h=dst_h, hsa_h=hsa_h,
                          hsb_h=hsb_h, ex_h=ex_h, da_h=da_h):
                    idxs, idxd, hsb, exb, dab = bufs[p]
                    sem = sems[p]
                    i1 = pltpu.async_copy(src_h.at[pl.ds(base, K)], idxs, sem)
                    i2 = pltpu.async_copy(dst_h.at[pl.ds(base, K)], idxd, sem)
                    i1.wait(); i2.wait()

                    @pl.when(c == 0)
                    def _():
                        pltpu.async_copy(hsa_h.at[idxs], hsb, sem)

                    @pl.when(c == 1)
                    def _():
                        pltpu.async_copy(hsb_h.at[idxs], hsb, sem)
                    pltpu.async_copy(da_h.at[idxd], dab, sem)
                    pltpu.async_copy(ex_h.at[pl.ds(base, K)], exb, sem)

                def finish(p, hsa_h=hsa_h, ex_h=ex_h, da_h=da_h, acc=acc):
                    idxs, idxd, hsb, exb, dab = bufs[p]
                    sem = sems[p]
                    pltpu.make_async_copy(hsa_h.at[idxs], hsb, sem).wait()
                    pltpu.make_async_copy(da_h.at[idxd], dab, sem).wait()
                    pltpu.make_async_copy(ex_h.at[pl.ds(0, K)], exb, sem).wait()

                    def msg_row(i, _):
                        m0 = jnp.zeros((16,), jnp.float32)
                        m1 = jnp.zeros((16,), jnp.float32)
                        av = exb[i, :] / ((dab[i, :] + 1e-16) * float(H))
                        for h in range(H):
                            a = av[h]
                            m0 = m0 + a * hsb[i, pl.ds(h * HALF, 16)]
                            m1 = m1 + a * hsb[i, pl.ds(h * HALF + 16, 16)]
                        mb[i, pl.ds(0, 16)] = m0
                        mb[i, pl.ds(16, 16)] = m1
                        return 0
                    lax.fori_loop(0, K, msg_row, 0)
                    pltpu.sync_copy(mb, acc.at[idxd], add=True)

                nb = (nfull - 1 - s) // NTILES + 1

                def base_of(k):
                    return (s + k * NTILES) * K

                start(base_of(0), 0)

                def kstep(k, _, start=start, finish=finish):
                    for p in range(2):
                        @pl.when(k % 2 == p)
                        def _(p=p):
                            @pl.when(k + 1 < nb)
                            def _():
                                start(base_of(k + 1), 1 - p)
                            finish(p)
                    return 0
                lax.fori_loop(0, nb, kstep, 0)

        plsc.subcore_barrier()
        for gi, (nd, _) in enumerate(groups):
            acc = accs[gi]
            out_h = outs[gi]

            def dump(r0, nr, acc=acc, out_h=out_h):
                @pl.when(c == 0)
                def _():
                    pltpu.sync_copy(acc.at[pl.ds(r0, nr)],
                                    out_h.at[pl.ds(r0, nr), pl.ds(0, HALF)])

                @pl.when(c == 1)
                def _():
                    pltpu.sync_copy(acc.at[pl.ds(r0, nr)],
                                    out_h.at[pl.ds(r0, nr), pl.ds(HALF, HALF)])
            _per_tile_rows(s, nd, dump)

    mesh = plsc.VectorSubcoreMesh(core_axis_name='c', subcore_axis_name='s')
    flat_in = [a for _, rels in groups for r in rels for a in r]
    outs = pl.kernel(body, out_type=out_type, mesh=mesh,
                     compiler_params=pltpu.CompilerParams(
                         use_tc_tiling_on_sc=False),
                     scratch_types=scratch)(*flat_in)
    return list(outs) if isinstance(outs, (list, tuple)) else [outs]


def _tc_pool(y, batch, bsum, block_r=512):
    n = y.shape[0]
    n_pad = pl.cdiv(n, block_r) * block_r
    y = jnp.pad(y, ((0, n_pad - n), (0, 0)))
    batch3 = jnp.pad(batch, (0, n_pad - n), constant_values=B).reshape(
        n_pad // block_r, 1, block_r)

    def body(y_ref, b_ref, bs_ref, s_ref, c_ref):
        i = pl.program_id(0)

        @pl.when(i == 0)
        def _():
            s_ref[...] = jnp.zeros_like(s_ref)
            c_ref[...] = jnp.zeros_like(c_ref)

        yb = y_ref[...] + bs_ref[...]
        ye = jnp.where(yb > 0, yb, jnp.exp(jnp.minimum(yb, 0.0)) - 1.0)
        ids = b_ref[0, 0, :]
        oh = (lax.broadcasted_iota(jnp.int32, (B, block_r), 0)
              == ids[None, :]).astype(jnp.float32)
        s_ref[...] += jnp.dot(oh, ye, preferred_element_type=jnp.float32)
        c_ref[...] = c_ref[...] + jnp.sum(oh, axis=1, keepdims=True)

    return pl.pallas_call(
        body,
        grid=(n_pad // block_r,),
        in_specs=[pl.BlockSpec((block_r, C), lambda i: (i, 0)),
                  pl.BlockSpec((1, 1, block_r), lambda i: (i, 0, 0)),
                  pl.BlockSpec((1, C), lambda i: (0, 0))],
        out_specs=[pl.BlockSpec((B, C), lambda i: (0, 0)),
                   pl.BlockSpec((B, C), lambda i: (0, 0))],
        out_shape=[jax.ShapeDtypeStruct((B, C), jnp.float32),
                   jax.ShapeDtypeStruct((B, C), jnp.float32)],
    )(y, batch3, bsum)


def _tc_head(pooled, lin_w, lin_b):
    def body(sc, cc, sv, cv, sk, ck, w_ref, b_ref, o_ref):
        z = jnp.concatenate(
            [sc[...] / jnp.maximum(cc[...], 1.0),
             sv[...] / jnp.maximum(cv[...], 1.0),
             sk[...] / jnp.maximum(ck[...], 1.0)], axis=1)
        logits = jnp.dot(z, w_ref[...], preferred_element_type=jnp.float32)
        logits = logits + b_ref[...]
        m = jnp.max(logits, axis=1, keepdims=True)
        e = jnp.exp(logits - m)
        o_ref[...] = (logits - m) - jnp.log(jnp.sum(e, axis=1, keepdims=True))

    args = [a for sc_cc in pooled for a in sc_cc] + [lin_w, lin_b.reshape(1, OUT)]
    return pl.pallas_call(
        body,
        out_shape=jax.ShapeDtypeStruct((B, OUT), jnp.float32),
    )(*args)


def kernel(x_control, x_variable, x_constant, params, edge_index_cc,
           edge_index_call, edge_index_cv, edge_index_vc, edge_index_kc,
           edge_index_ck, batch_control, batch_variable, batch_constant):
    eis = {'cc': edge_index_cc, 'call': edge_index_call, 'cv': edge_index_cv,
           'vc': edge_index_vc, 'kc': edge_index_kc, 'ck': edge_index_ck}
    batches = {'control': batch_control, 'variable': batch_variable,
               'constant': batch_constant}
    xd = {'control': x_control, 'variable': x_variable, 'constant': x_constant}
    src_t = {name: s for (s, _, name) in RELS}
    dst_t = {name: d for (_, d, name) in RELS}

    bsum = None
    for li, d_in in (('layer0', 128), ('layer1', C)):
        lp = params[li]
        wcat, widths = _build_wcat(lp, d_in)
        hsA, hsB, aS, aD = {}, {}, {}, {}
        bs_t = {t: (None if bsum is None else bsum[t]) for t in NTYPES}
        for t in NTYPES:
            aouts = _tc_matmul(xd[t], wcat[t][0], widths[t][0], bsum=bs_t[t])
            src_rels = [name for (s, _, name) in RELS if s == t]
            dst_rels = [name for (_, dt, name) in RELS if dt == t]
            for i, name in enumerate(src_rels):
                aS[name] = aouts[i]
            for i, name in enumerate(dst_rels):
                aD[name] = aouts[len(src_rels) + i]

        ex, dnm = {}, {}
        for pair in (['cc', 'vc'], ['call', 'kc'], ['cv', 'ck']):
            res = _sc_pass1([(eis[n][0], eis[n][1], aS[n], aD[n])
                             for n in pair])
            for n, (e_, d_) in zip(pair, res):
                ex[n], dnm[n] = e_, d_

        for t in NTYPES:
            houts = _tc_matmul(xd[t], wcat[t][1], widths[t][1], bsum=bs_t[t])
            src_rels = [name for (s, _, name) in RELS if s == t]
            for i, name in enumerate(src_rels):
                hsA[name], hsB[name] = houts[2 * i], houts[2 * i + 1]

        def rel_args(n):
            return (eis[n][0], eis[n][1], hsA[n], hsB[n], ex[n], dnm[n])

        (res_c,) = _sc_pass2([(xd['control'].shape[0],
                               [rel_args(n) for n in ('cc', 'call', 'vc', 'kc')])],
                             K=32, zr=64)
        res_v, res_k = _sc_pass2([
            (xd['variable'].shape[0], [rel_args('cv')]),
            (xd['constant'].shape[0], [rel_args('ck')])])

        nxt, bsum_n = {}, {}
        for t, yt in zip(NTYPES, (res_c, res_v, res_k)):
            nxt[t] = yt
            bs = sum(lp[n]['bias'] for n in eis if dst_t[n] == t)
            bsum_n[t] = bs.reshape(1, C)
        xd, bsum = nxt, bsum_n

    pooled = [_tc_pool(xd[t], batches[t], bsum[t]) for t in NTYPES]
    return _tc_head(pooled, params['lin_w'], params['lin_b'])


# speedup vs baseline: 1.0334x; 1.0334x over previous
import functools

import jax
import jax.numpy as jnp
from jax import lax
from jax.experimental import pallas as pl
from jax.experimental.pallas import tpu as pltpu
from jax.experimental.pallas import tpu_sc as plsc

H = 8
C = 64
HALF = C // 2
AW = 16
B = 64
OUT = 10
RELS = [('control', 'control', 'cc'), ('control', 'control', 'call'),
        ('control', 'variable', 'cv'), ('variable', 'control', 'vc'),
        ('constant', 'control', 'kc'), ('control', 'constant', 'ck')]
NTYPES = ('control', 'variable', 'constant')
NSC = 2
NTILES = 16
ZR = 128


def _tc_matmul(x, w, widths, bsum=None, block_r=512):
    n, d = x.shape
    tot = w.shape[1]
    nin = 2 + (1 if bsum is not None else 0)

    def body(*refs):
        xb = refs[0][...]
        if bsum is not None:
            xb = xb + refs[2][...]
            xb = jnp.where(xb > 0, xb, jnp.exp(jnp.minimum(xb, 0.0)) - 1.0)
        acc = jnp.dot(xb, refs[1][...], preferred_element_type=jnp.float32)
        off = 0
        for o_ref in refs[nin:]:
            wd = o_ref.shape[1]
            o_ref[...] = acc[:, off:off + wd]
            off += wd

    in_specs = [pl.BlockSpec((block_r, d), lambda i: (i, 0)),
                pl.BlockSpec((d, tot), lambda i: (0, 0))]
    args = [x, w]
    if bsum is not None:
        in_specs.append(pl.BlockSpec((1, d), lambda i: (0, 0)))
        args.append(bsum)
    return pl.pallas_call(
        body,
        grid=(pl.cdiv(n, block_r),),
        in_specs=in_specs,
        out_specs=[pl.BlockSpec((block_r, wd), lambda i: (i, 0)) for wd in widths],
        out_shape=[jax.ShapeDtypeStruct((n, wd), jnp.float32) for wd in widths],
    )(*args)


def _build_wcat(lp, d_in):
    perm_a = [h * C + c for h in range(H) for c in range(HALF)]
    perm_b = [h * C + HALF + c for h in range(H) for c in range(HALF)]
    wcat, widths = {}, {}
    for t in NTYPES:
        src_rels = [name for (s, _, name) in RELS if s == t]
        dst_rels = [name for (_, dt, name) in RELS if dt == t]
        acols, awd = [], []
        for name in src_rels:
            p = lp[name]
            ws2 = jnp.einsum('dhc,hc->dh', p['w_src'].reshape(d_in, H, C),
                             p['att_src'][0])
            acols.append(jnp.pad(ws2, ((0, 0), (0, AW - H)))); awd.append(AW)
        for name in dst_rels:
            p = lp[name]
            wd2 = jnp.einsum('dhc,hc->dh', p['w_dst'].reshape(d_in, H, C),
                             p['att_dst'][0])
            acols.append(jnp.pad(wd2, ((0, 0), (0, AW - H)))); awd.append(AW)
        hcols, hwd = [], []
        for name in src_rels:
            w = lp[name]['w_src']
            hcols.append(w[:, perm_a]); hwd.append(H * HALF)
            hcols.append(w[:, perm_b]); hwd.append(H * HALF)
        wcat[t] = (jnp.concatenate(acols, axis=1),
                   jnp.concatenate(hcols, axis=1))
        widths[t] = (awd, hwd)
    return wcat, widths


def _tile_rows(nd):
    r8 = -(-(nd // NTILES) // 8) * 8
    last = nd - (NTILES - 1) * r8
    assert last >= 0 and last % 8 == 0
    return r8, last


def _zero_rows(acc, row0, nrows, zbuf):
    zr = zbuf.shape[0]
    nz, rem = nrows // zr, nrows % zr

    def zk(k, _):
        pltpu.sync_copy(zbuf, acc.at[pl.ds(row0 + k * zr, zr)])
        return 0
    lax.fori_loop(0, nz, zk, 0)
    if rem:
        pltpu.sync_copy(zbuf.at[pl.ds(0, rem)],
                        acc.at[pl.ds(row0 + nz * zr, rem)])


def _per_tile_rows(s, acc_nd, fn):
    r8, last = _tile_rows(acc_nd)

    @pl.when(s < NTILES - 1)
    def _():
        fn(s * r8, r8)

    @pl.when(s == NTILES - 1)
    def _():
        fn((NTILES - 1) * r8, last)


def _fill_zeros(zbuf):
    z = jnp.zeros((16,), jnp.float32)

    def zrow(i, _):
        for w in range(zbuf.shape[1] // 16):
            zbuf[i, pl.ds(w * 16, 16)] = z
        return 0
    lax.fori_loop(0, zbuf.shape[0], zrow, 0)


def _sc_pass1(rels):
    nrel = len(rels)
    nr = nrel // NSC
    assert nr * NSC == nrel
    E = rels[0][0].shape[0]
    K = 80
    nfull = E // K
    assert nfull * K == E and K % 8 == 0
    nds = [r[3].shape[0] for r in rels]
    acc_nds = [max(nds[j], nds[nr + j]) for j in range(nr)]

    out_type = []
    for nd in nds:
        out_type += [jax.ShapeDtypeStruct((E, AW), jnp.float32),
                     jax.ShapeDtypeStruct((nd, AW), jnp.float32)]
    p1set = [pltpu.VMEM((K,), jnp.int32), pltpu.VMEM((K,), jnp.int32),
             pltpu.VMEM((K, AW), jnp.float32), pltpu.VMEM((K, AW), jnp.float32)]
    scratch = (p1set + p1set
               + [pltpu.VMEM((K, AW), jnp.float32),
                  pltpu.VMEM((ZR, AW), jnp.float32),
                  pltpu.SemaphoreType.DMA, pltpu.SemaphoreType.DMA])
    scratch += [pltpu.VMEM_SHARED((nd, AW), jnp.float32) for nd in acc_nds]

    def body(*refs):
        ins = refs[:4 * nrel]
        outs = refs[4 * nrel:4 * nrel + 2 * nrel]
        scr = refs[4 * nrel + 2 * nrel:]
        bufs = (scr[0:4], scr[4:8])
        exb, zbuf = scr[8:10]
        sems = (scr[10], scr[11])
        accs = scr[12:]

        c = lax.axis_index('c')
        s = lax.axis_index('s')
        _fill_zeros(zbuf)

        def process(r, acc):
            src_h, dst_h, as_h, ad_h = ins[4 * r:4 * r + 4]
            ex_h = outs[2 * r]

            def start(base, p):
                bi_s, bi_d, ba_s, ba_d = bufs[p]
                sem = sems[p]
                i1 = pltpu.async_copy(src_h.at[pl.ds(base, K)], bi_s, sem)
                i2 = pltpu.async_copy(dst_h.at[pl.ds(base, K)], bi_d, sem)
                i1.wait(); i2.wait()
                pltpu.async_copy(as_h.at[bi_s], ba_s, sem)
                pltpu.async_copy(ad_h.at[bi_d], ba_d, sem)

            def finish(base, p):
                bi_s, bi_d, ba_s, ba_d = bufs[p]
                sem = sems[p]
                pltpu.make_async_copy(as_h.at[bi_s], ba_s, sem).wait()
                pltpu.make_async_copy(ad_h.at[bi_d], ba_d, sem).wait()

                def row(i, _):
                    v = ba_s[i, :] + ba_d[i, :]
                    v = jnp.where(v > 0, v, 0.2 * v)
                    exb[i, :] = jnp.exp(v)
                    return 0
                lax.fori_loop(0, K, row, 0)
                pltpu.sync_copy(exb, acc.at[bi_d], add=True)
                pltpu.sync_copy(exb, ex_h.at[pl.ds(base, K)])

            nb = (nfull - 1 - s) // NTILES + 1

            def base_of(k):
                return (s + k * NTILES) * K

            start(base_of(0), 0)

            def kstep(k, _):
                for p in range(2):
                    @pl.when(k % 2 == p)
                    def _(p=p):
                        @pl.when(k + 1 < nb)
                        def _():
                            start(base_of(k + 1), 1 - p)
                        finish(base_of(k), p)
                return 0
            lax.fori_loop(0, nb, kstep, 0)

        for sc in range(NSC):
            @pl.when(c == sc)
            def _(sc=sc):
                for j in range(nr):
                    r = sc * nr + j
                    _per_tile_rows(s, nds[r],
                                   lambda r0, nrw, j=j: _zero_rows(
                                       accs[j], r0, nrw, zbuf))
        plsc.subcore_barrier()

        for sc in range(NSC):
            @pl.when(c == sc)
            def _(sc=sc):
                for j in range(nr):
                    process(sc * nr + j, accs[j])
        plsc.subcore_barrier()

        for sc in range(NSC):
            @pl.when(c == sc)
            def _(sc=sc):
                for j in range(nr):
                    r = sc * nr + j
                    dn = outs[2 * r + 1]

                    def dump(r0, nrw, j=j, dn=dn):
                        pltpu.sync_copy(accs[j].at[pl.ds(r0, nrw)],
                                        dn.at[pl.ds(r0, nrw)])
                    _per_tile_rows(s, nds[r], dump)

    mesh = plsc.VectorSubcoreMesh(core_axis_name='c', subcore_axis_name='s')
    flat_in = [a for r in rels for a in r]
    outs = pl.kernel(body, out_type=out_type, mesh=mesh,
                     compiler_params=pltpu.CompilerParams(
                         use_tc_tiling_on_sc=False),
                     scratch_types=scratch)(*flat_in)
    return [tuple(outs[2 * r:2 * r + 2]) for r in range(nrel)]


def _sc_pass2(groups, K=64, zr=ZR):
    E = groups[0][1][0][0].shape[0]
    nfull = E // K
    assert nfull * K == E

    out_type = []
    for nd, _ in groups:
        out_type += [jax.ShapeDtypeStruct((nd, HALF), jnp.float32),
                     jax.ShapeDtypeStruct((nd, HALF), jnp.float32)]
    bufset = [pltpu.VMEM((K,), jnp.int32), pltpu.VMEM((K,), jnp.int32),
              pltpu.VMEM((K, H * HALF), jnp.float32),
              pltpu.VMEM((K, AW), jnp.float32),
              pltpu.VMEM((K, AW), jnp.float32)]
    scratch = (bufset + bufset
               + [pltpu.VMEM((K, AW), jnp.float32),
                  pltpu.VMEM((K, HALF), jnp.float32),
                  pltpu.VMEM((zr, HALF), jnp.float32),
                  pltpu.SemaphoreType.DMA, pltpu.SemaphoreType.DMA])
    scratch += [pltpu.VMEM_SHARED((nd, HALF), jnp.float32) for nd, _ in groups]

    nin = sum(6 * len(rels) for _, rels in groups)

    def body(*refs):
        ins = refs[:nin]
        outs = refs[nin:nin + 2 * len(groups)]
        scr = refs[nin + 2 * len(groups):]
        bufs = (scr[0:5], scr[5:10])
        attb, mb, zbuf = scr[10:13]
        sems = (scr[13], scr[14])
        accs = scr[15:]

        c = lax.axis_index('c')
        s = lax.axis_index('s')
        _fill_zeros(zbuf)
        for acc in accs:
            _per_tile_rows(s, acc.shape[0],
                           lambda r0, nr, acc=acc: _zero_rows(acc, r0, nr, zbuf))
        plsc.subcore_barrier()

        off = 0
        for gi, (nd, rels) in enumerate(groups):
            acc = accs[gi]
            for _ in rels:
                src_h, dst_h, hsa_h, hsb_h, ex_h, da_h = ins[off:off + 6]
                off += 6

                def start(base, p, src_h=src_h, dst_h=dst_h, hsa_h=hsa_h,
                          hsb_h=hsb_h, ex_h=ex_h, da_h=da_h):
                    idxs, idxd, hsb, exb, dab = bufs[p]
                    sem = sems[p]
                    i1 = pltpu.async_copy(src_h.at[pl.ds(base, K)], idxs, sem)
                    i2 = pltpu.async_copy(dst_h.at[pl.ds(base, K)], idxd, sem)
                    i1.wait(); i2.wait()

                    @pl.when(c == 0)
                    def _():
                        pltpu.async_copy(hsa_h.at[idxs], hsb, sem)

                    @pl.when(c == 1)
                    def _():
                        pltpu.async_copy(hsb_h.at[idxs], hsb, sem)
                    pltpu.async_copy(da_h.at[idxd], dab, sem)
                    pltpu.async_copy(ex_h.at[pl.ds(base, K)], exb, sem)

                def finish(p, hsa_h=hsa_h, ex_h=ex_h, da_h=da_h, acc=acc):
                    idxs, idxd, hsb, exb, dab = bufs[p]
                    sem = sems[p]
                    pltpu.make_async_copy(hsa_h.at[idxs], hsb, sem).wait()
                    pltpu.make_async_copy(da_h.at[idxd], dab, sem).wait()
                    pltpu.make_async_copy(ex_h.at[pl.ds(0, K)], exb, sem).wait()

                    def att_row(i, _):
                        attb[i, :] = exb[i, :] / (
                            (dab[i, :] + 1e-16) * float(H))
                        return 0
                    lax.fori_loop(0, K, att_row, 0)

                    def msg_row(i, _):
                        m0 = jnp.zeros((16,), jnp.float32)
                        m1 = jnp.zeros((16,), jnp.float32)
                        av = attb[i, :]
                        for h in range(H):
                            a = av[h]
                            m0 = m0 + a * hsb[i, pl.ds(h * HALF, 16)]
                            m1 = m1 + a * hsb[i, pl.ds(h * HALF + 16, 16)]
                        mb[i, pl.ds(0, 16)] = m0
                        mb[i, pl.ds(16, 16)] = m1
                        return 0
                    lax.fori_loop(0, K, msg_row, 0)
                    pltpu.sync_copy(mb, acc.at[idxd], add=True)

                nb = (nfull - 1 - s) // NTILES + 1

                def base_of(k):
                    return (s + k * NTILES) * K

                start(base_of(0), 0)

                def kstep(k, _, start=start, finish=finish):
                    for p in range(2):
                        @pl.when(k % 2 == p)
                        def _(p=p):
                            @pl.when(k + 1 < nb)
                            def _():
                                start(base_of(k + 1), 1 - p)
                            finish(p)
                    return 0
                lax.fori_loop(0, nb, kstep, 0)

        plsc.subcore_barrier()
        for gi, (nd, _) in enumerate(groups):
            acc = accs[gi]
            oa, ob = outs[2 * gi], outs[2 * gi + 1]

            def dump(r0, nr, acc=acc, oa=oa, ob=ob):
                @pl.when(c == 0)
                def _():
                    pltpu.sync_copy(acc.at[pl.ds(r0, nr)], oa.at[pl.ds(r0, nr)])

                @pl.when(c == 1)
                def _():
                    pltpu.sync_copy(acc.at[pl.ds(r0, nr)], ob.at[pl.ds(r0, nr)])
            _per_tile_rows(s, nd, dump)

    mesh = plsc.VectorSubcoreMesh(core_axis_name='c', subcore_axis_name='s')
    flat_in = [a for _, rels in groups for r in rels for a in r]
    outs = pl.kernel(body, out_type=out_type, mesh=mesh,
                     compiler_params=pltpu.CompilerParams(
                         use_tc_tiling_on_sc=False),
                     scratch_types=scratch)(*flat_in)
    return [tuple(outs[2 * g:2 * g + 2]) for g in range(len(groups))]


def _tc_pool(y, batch, bsum, block_r=512):
    n = y.shape[0]
    n_pad = pl.cdiv(n, block_r) * block_r
    y = jnp.pad(y, ((0, n_pad - n), (0, 0)))
    batch3 = jnp.pad(batch, (0, n_pad - n), constant_values=B).reshape(
        n_pad // block_r, 1, block_r)

    def body(y_ref, b_ref, bs_ref, s_ref, c_ref):
        i = pl.program_id(0)

        @pl.when(i == 0)
        def _():
            s_ref[...] = jnp.zeros_like(s_ref)
            c_ref[...] = jnp.zeros_like(c_ref)

        yb = y_ref[...] + bs_ref[...]
        ye = jnp.where(yb > 0, yb, jnp.exp(jnp.minimum(yb, 0.0)) - 1.0)
        ids = b_ref[0, 0, :]
        oh = (lax.broadcasted_iota(jnp.int32, (B, block_r), 0)
              == ids[None, :]).astype(jnp.float32)
        s_ref[...] += jnp.dot(oh, ye, preferred_element_type=jnp.float32)
        c_ref[...] = c_ref[...] + jnp.sum(oh, axis=1, keepdims=True)

    return pl.pallas_call(
        body,
        grid=(n_pad // block_r,),
        in_specs=[pl.BlockSpec((block_r, C), lambda i: (i, 0)),
                  pl.BlockSpec((1, 1, block_r), lambda i: (i, 0, 0)),
                  pl.BlockSpec((1, C), lambda i: (0, 0))],
        out_specs=[pl.BlockSpec((B, C), lambda i: (0, 0)),
                   pl.BlockSpec((B, C), lambda i: (0, 0))],
        out_shape=[jax.ShapeDtypeStruct((B, C), jnp.float32),
                   jax.ShapeDtypeStruct((B, C), jnp.float32)],
    )(y, batch3, bsum)


def _tc_head(pooled, lin_w, lin_b):
    def body(sc, cc, sv, cv, sk, ck, w_ref, b_ref, o_ref):
        z = jnp.concatenate(
            [sc[...] / jnp.maximum(cc[...], 1.0),
             sv[...] / jnp.maximum(cv[...], 1.0),
             sk[...] / jnp.maximum(ck[...], 1.0)], axis=1)
        logits = jnp.dot(z, w_ref[...], preferred_element_type=jnp.float32)
        logits = logits + b_ref[...]
        m = jnp.max(logits, axis=1, keepdims=True)
        e = jnp.exp(logits - m)
        o_ref[...] = (logits - m) - jnp.log(jnp.sum(e, axis=1, keepdims=True))

    args = [a for sc_cc in pooled for a in sc_cc] + [lin_w, lin_b.reshape(1, OUT)]
    return pl.pallas_call(
        body,
        out_shape=jax.ShapeDtypeStruct((B, OUT), jnp.float32),
    )(*args)


def kernel(x_control, x_variable, x_constant, params, edge_index_cc,
           edge_index_call, edge_index_cv, edge_index_vc, edge_index_kc,
           edge_index_ck, batch_control, batch_variable, batch_constant):
    eis = {'cc': edge_index_cc, 'call': edge_index_call, 'cv': edge_index_cv,
           'vc': edge_index_vc, 'kc': edge_index_kc, 'ck': edge_index_ck}
    batches = {'control': batch_control, 'variable': batch_variable,
               'constant': batch_constant}
    xd = {'control': x_control, 'variable': x_variable, 'constant': x_constant}
    src_t = {name: s for (s, _, name) in RELS}
    dst_t = {name: d for (_, d, name) in RELS}

    bsum = None
    for li, d_in in (('layer0', 128), ('layer1', C)):
        lp = params[li]
        wcat, widths = _build_wcat(lp, d_in)
        hsA, hsB, aS, aD = {}, {}, {}, {}
        bs_t = {t: (None if bsum is None else bsum[t]) for t in NTYPES}
        for t in NTYPES:
            aouts = _tc_matmul(xd[t], wcat[t][0], widths[t][0], bsum=bs_t[t])
            src_rels = [name for (s, _, name) in RELS if s == t]
            dst_rels = [name for (_, dt, name) in RELS if dt == t]
            for i, name in enumerate(src_rels):
                aS[name] = aouts[i]
            for i, name in enumerate(dst_rels):
                aD[name] = aouts[len(src_rels) + i]

        ex, dnm = {}, {}
        for pair in (['cc', 'vc'], ['call', 'kc'], ['cv', 'ck']):
            res = _sc_pass1([(eis[n][0], eis[n][1], aS[n], aD[n])
                             for n in pair])
            for n, (e_, d_) in zip(pair, res):
                ex[n], dnm[n] = e_, d_

        for t in NTYPES:
            houts = _tc_matmul(xd[t], wcat[t][1], widths[t][1], bsum=bs_t[t])
            src_rels = [name for (s, _, name) in RELS if s == t]
            for i, name in enumerate(src_rels):
                hsA[name], hsB[name] = houts[2 * i], houts[2 * i + 1]

        def rel_args(n):
            return (eis[n][0], eis[n][1], hsA[n], hsB[n], ex[n], dnm[n])

        (res_c,) = _sc_pass2([(xd['control'].shape[0],
                               [rel_args(n) for n in ('cc', 'call', 'vc', 'kc')])],
                             K=32, zr=64)
        res_v, res_k = _sc_pass2([
            (xd['variable'].shape[0], [rel_args('cv')]),
            (xd['constant'].shape[0], [rel_args('ck')])])

        nxt, bsum_n = {}, {}
        for t, (oa, ob) in zip(NTYPES, (res_c, res_v, res_k)):
            nxt[t] = jnp.concatenate([oa, ob], axis=1)
            bs = sum(lp[n]['bias'] for n in eis if dst_t[n] == t)
            bsum_n[t] = bs.reshape(1, C)
        xd, bsum = nxt, bsum_n

    pooled = [_tc_pool(xd[t], batches[t], bsum[t]) for t in NTYPES]
    return _tc_head(pooled, params['lin_w'], params['lin_b'])


# speedup vs baseline: 1.0570x; 1.0228x over previous
import functools

import jax
import jax.numpy as jnp
from jax import lax
from jax.experimental import pallas as pl
from jax.experimental.pallas import tpu as pltpu
from jax.experimental.pallas import tpu_sc as plsc

H = 8
C = 64
HALF = C // 2
AW = 16
B = 64
OUT = 10
RELS = [('control', 'control', 'cc'), ('control', 'control', 'call'),
        ('control', 'variable', 'cv'), ('variable', 'control', 'vc'),
        ('constant', 'control', 'kc'), ('control', 'constant', 'ck')]
NTYPES = ('control', 'variable', 'constant')
NSC = 2
NTILES = 16
ZR = 128


def _tc_matmul(x, w, widths, bsum=None, block_r=512):
    n, d = x.shape
    tot = w.shape[1]
    nin = 2 + (1 if bsum is not None else 0)

    def body(*refs):
        xb = refs[0][...]
        if bsum is not None:
            xb = xb + refs[2][...]
            xb = jnp.where(xb > 0, xb, jnp.exp(jnp.minimum(xb, 0.0)) - 1.0)
        acc = jnp.dot(xb, refs[1][...], preferred_element_type=jnp.float32)
        off = 0
        for o_ref in refs[nin:]:
            wd = o_ref.shape[1]
            o_ref[...] = acc[:, off:off + wd]
            off += wd

    in_specs = [pl.BlockSpec((block_r, d), lambda i: (i, 0)),
                pl.BlockSpec((d, tot), lambda i: (0, 0))]
    args = [x, w]
    if bsum is not None:
        in_specs.append(pl.BlockSpec((1, d), lambda i: (0, 0)))
        args.append(bsum)
    return pl.pallas_call(
        body,
        grid=(pl.cdiv(n, block_r),),
        in_specs=in_specs,
        out_specs=[pl.BlockSpec((block_r, wd), lambda i: (i, 0)) for wd in widths],
        out_shape=[jax.ShapeDtypeStruct((n, wd), jnp.float32) for wd in widths],
    )(*args)


def _build_wcat(lp, d_in):
    perm_a = [h * C + c for h in range(H) for c in range(HALF)]
    perm_b = [h * C + HALF + c for h in range(H) for c in range(HALF)]
    wcat, widths = {}, {}
    for t in NTYPES:
        src_rels = [name for (s, _, name) in RELS if s == t]
        dst_rels = [name for (_, dt, name) in RELS if dt == t]
        acols, awd = [], []
        for name in src_rels:
            p = lp[name]
            ws2 = jnp.einsum('dhc,hc->dh', p['w_src'].reshape(d_in, H, C),
                             p['att_src'][0])
            acols.append(jnp.pad(ws2, ((0, 0), (0, AW - H)))); awd.append(AW)
        for name in dst_rels:
            p = lp[name]
            wd2 = jnp.einsum('dhc,hc->dh', p['w_dst'].reshape(d_in, H, C),
                             p['att_dst'][0])
            acols.append(jnp.pad(wd2, ((0, 0), (0, AW - H)))); awd.append(AW)
        hcols, hwd = [], []
        for name in src_rels:
            w = lp[name]['w_src']
            hcols.append(w[:, perm_a]); hwd.append(H * HALF)
            hcols.append(w[:, perm_b]); hwd.append(H * HALF)
        wcat[t] = (jnp.concatenate(acols, axis=1),
                   jnp.concatenate(hcols, axis=1))
        widths[t] = (awd, hwd)
    return wcat, widths


def _tile_rows(nd):
    r8 = -(-(nd // NTILES) // 8) * 8
    last = nd - (NTILES - 1) * r8
    assert last >= 0 and last % 8 == 0
    return r8, last


def _zero_rows(acc, row0, nrows, zbuf):
    zr = zbuf.shape[0]
    nz, rem = nrows // zr, nrows % zr

    def zk(k, _):
        pltpu.sync_copy(zbuf, acc.at[pl.ds(row0 + k * zr, zr)])
        return 0
    lax.fori_loop(0, nz, zk, 0)
    if rem:
        pltpu.sync_copy(zbuf.at[pl.ds(0, rem)],
                        acc.at[pl.ds(row0 + nz * zr, rem)])


def _per_tile_rows(s, acc_nd, fn):
    r8, last = _tile_rows(acc_nd)

    @pl.when(s < NTILES - 1)
    def _():
        fn(s * r8, r8)

    @pl.when(s == NTILES - 1)
    def _():
        fn((NTILES - 1) * r8, last)


def _fill_zeros(zbuf):
    z = jnp.zeros((16,), jnp.float32)

    def zrow(i, _):
        for w in range(zbuf.shape[1] // 16):
            zbuf[i, pl.ds(w * 16, 16)] = z
        return 0
    lax.fori_loop(0, zbuf.shape[0], zrow, 0)


def _sc_pass1(rels):
    nrel = len(rels)
    nr = nrel // NSC
    assert nr * NSC == nrel
    E = rels[0][0].shape[0]
    K = 80
    nfull = E // K
    assert nfull * K == E and K % 8 == 0
    nds = [r[3].shape[0] for r in rels]
    acc_nds = [max(nds[j], nds[nr + j]) for j in range(nr)]

    out_type = []
    for nd in nds:
        out_type += [jax.ShapeDtypeStruct((E, AW), jnp.float32),
                     jax.ShapeDtypeStruct((nd, AW), jnp.float32)]
    p1set = [pltpu.VMEM((K,), jnp.int32), pltpu.VMEM((K,), jnp.int32),
             pltpu.VMEM((K, AW), jnp.float32), pltpu.VMEM((K, AW), jnp.float32)]
    scratch = (p1set + p1set
               + [pltpu.VMEM((K, AW), jnp.float32),
                  pltpu.VMEM((ZR, AW), jnp.float32),
                  pltpu.SemaphoreType.DMA, pltpu.SemaphoreType.DMA])
    scratch += [pltpu.VMEM_SHARED((nd, AW), jnp.float32) for nd in acc_nds]

    def body(*refs):
        ins = refs[:4 * nrel]
        outs = refs[4 * nrel:4 * nrel + 2 * nrel]
        scr = refs[4 * nrel + 2 * nrel:]
        bufs = (scr[0:4], scr[4:8])
        exb, zbuf = scr[8:10]
        sems = (scr[10], scr[11])
        accs = scr[12:]

        c = lax.axis_index('c')
        s = lax.axis_index('s')
        _fill_zeros(zbuf)

        def process(r, acc):
            src_h, dst_h, as_h, ad_h = ins[4 * r:4 * r + 4]
            ex_h = outs[2 * r]

            def start(base, p):
                bi_s, bi_d, ba_s, ba_d = bufs[p]
                sem = sems[p]
                i1 = pltpu.async_copy(src_h.at[pl.ds(base, K)], bi_s, sem)
                i2 = pltpu.async_copy(dst_h.at[pl.ds(base, K)], bi_d, sem)
                i1.wait(); i2.wait()
                pltpu.async_copy(as_h.at[bi_s], ba_s, sem)
                pltpu.async_copy(ad_h.at[bi_d], ba_d, sem)

            def finish(base, p):
                bi_s, bi_d, ba_s, ba_d = bufs[p]
                sem = sems[p]
                pltpu.make_async_copy(as_h.at[bi_s], ba_s, sem).wait()
                pltpu.make_async_copy(ad_h.at[bi_d], ba_d, sem).wait()

                def row(i, _):
                    v = ba_s[i, :] + ba_d[i, :]
                    v = jnp.where(v > 0, v, 0.2 * v)
                    exb[i, :] = jnp.exp(v)
                    return 0
                lax.fori_loop(0, K, row, 0)
                pltpu.sync_copy(exb, acc.at[bi_d], add=True)
                pltpu.sync_copy(exb, ex_h.at[pl.ds(base, K)])

            nb = (nfull - 1 - s) // NTILES + 1

            def base_of(k):
                return (s + k * NTILES) * K

            start(base_of(0), 0)

            def kstep(k, _):
                for p in range(2):
                    @pl.when(k % 2 == p)
                    def _(p=p):
                        @pl.when(k + 1 < nb)
                        def _():
                            start(base_of(k + 1), 1 - p)
                        finish(base_of(k), p)
                return 0
            lax.fori_loop(0, nb, kstep, 0)

        for sc in range(NSC):
            @pl.when(c == sc)
            def _(sc=sc):
                for j in range(nr):
                    r = sc * nr + j
                    _per_tile_rows(s, nds[r],
                                   lambda r0, nrw, j=j: _zero_rows(
                                       accs[j], r0, nrw, zbuf))
        plsc.subcore_barrier()

        for sc in range(NSC):
            @pl.when(c == sc)
            def _(sc=sc):
                for j in range(nr):
                    process(sc * nr + j, accs[j])
        plsc.subcore_barrier()

        for sc in range(NSC):
            @pl.when(c == sc)
            def _(sc=sc):
                for j in range(nr):
                    r = sc * nr + j
                    dn = outs[2 * r + 1]

                    def dump(r0, nrw, j=j, dn=dn):
                        pltpu.sync_copy(accs[j].at[pl.ds(r0, nrw)],
                                        dn.at[pl.ds(r0, nrw)])
                    _per_tile_rows(s, nds[r], dump)

    mesh = plsc.VectorSubcoreMesh(core_axis_name='c', subcore_axis_name='s')
    flat_in = [a for r in rels for a in r]
    outs = pl.kernel(body, out_type=out_type, mesh=mesh,
                     compiler_params=pltpu.CompilerParams(
                         use_tc_tiling_on_sc=False),
                     scratch_types=scratch)(*flat_in)
    return [tuple(outs[2 * r:2 * r + 2]) for r in range(nrel)]


def _sc_pass2(groups, K=64, zr=ZR):
    E = groups[0][1][0][0].shape[0]
    nfull = E // K
    assert nfull * K == E

    out_type = []
    for nd, _ in groups:
        out_type += [jax.ShapeDtypeStruct((nd, HALF), jnp.float32),
                     jax.ShapeDtypeStruct((nd, HALF), jnp.float32)]
    bufset = [pltpu.VMEM((K,), jnp.int32), pltpu.VMEM((K,), jnp.int32),
              pltpu.VMEM((K, H * HALF), jnp.float32),
              pltpu.VMEM((K, AW), jnp.float32),
              pltpu.VMEM((K, AW), jnp.float32)]
    scratch = (bufset + bufset
               + [pltpu.VMEM((K, AW), jnp.float32),
                  pltpu.VMEM((K, HALF), jnp.float32),
                  pltpu.VMEM((zr, HALF), jnp.float32),
                  pltpu.SemaphoreType.DMA, pltpu.SemaphoreType.DMA])
    scratch += [pltpu.VMEM_SHARED((nd, HALF), jnp.float32) for nd, _ in groups]

    nin = sum(6 * len(rels) for _, rels in groups)

    def body(*refs):
        ins = refs[:nin]
        outs = refs[nin:nin + 2 * len(groups)]
        scr = refs[nin + 2 * len(groups):]
        bufs = (scr[0:5], scr[5:10])
        attb, mb, zbuf = scr[10:13]
        sems = (scr[13], scr[14])
        accs = scr[15:]

        c = lax.axis_index('c')
        s = lax.axis_index('s')
        _fill_zeros(zbuf)
        for acc in accs:
            _per_tile_rows(s, acc.shape[0],
                           lambda r0, nr, acc=acc: _zero_rows(acc, r0, nr, zbuf))
        plsc.subcore_barrier()

        off = 0
        for gi, (nd, rels) in enumerate(groups):
            acc = accs[gi]
            for _ in rels:
                src_h, dst_h, hsa_h, hsb_h, ex_h, da_h = ins[off:off + 6]
                off += 6

                def start(base, p, src_h=src_h, dst_h=dst_h, hsa_h=hsa_h,
                          hsb_h=hsb_h, ex_h=ex_h, da_h=da_h):
                    idxs, idxd, hsb, exb, dab = bufs[p]
                    sem = sems[p]
                    i1 = pltpu.async_copy(src_h.at[pl.ds(base, K)], idxs, sem)
                    i2 = pltpu.async_copy(dst_h.at[pl.ds(base, K)], idxd, sem)
                    i1.wait(); i2.wait()

                    @pl.when(c == 0)
                    def _():
                        pltpu.async_copy(hsa_h.at[idxs], hsb, sem)

                    @pl.when(c == 1)
                    def _():
                        pltpu.async_copy(hsb_h.at[idxs], hsb, sem)
                    pltpu.async_copy(da_h.at[idxd], dab, sem)
                    pltpu.async_copy(ex_h.at[pl.ds(base, K)], exb, sem)

                def finish(p, hsa_h=hsa_h, ex_h=ex_h, da_h=da_h, acc=acc):
                    idxs, idxd, hsb, exb, dab = bufs[p]
                    sem = sems[p]
                    pltpu.make_async_copy(hsa_h.at[idxs], hsb, sem).wait()
                    pltpu.make_async_copy(da_h.at[idxd], dab, sem).wait()
                    pltpu.make_async_copy(ex_h.at[pl.ds(0, K)], exb, sem).wait()

                    def att_row(i, _):
                        attb[i, :] = exb[i, :] / (
                            (dab[i, :] + 1e-16) * float(H))
                        return 0
                    lax.fori_loop(0, K, att_row, 0)

                    def msg_row(i, _):
                        m0 = jnp.zeros((16,), jnp.float32)
                        m1 = jnp.zeros((16,), jnp.float32)
                        av = attb[i, :]
                        for h in range(H):
                            a = av[h]
                            m0 = m0 + a * hsb[i, pl.ds(h * HALF, 16)]
                            m1 = m1 + a * hsb[i, pl.ds(h * HALF + 16, 16)]
                        mb[i, pl.ds(0, 16)] = m0
                        mb[i, pl.ds(16, 16)] = m1
                        return 0
                    lax.fori_loop(0, K, msg_row, 0)
                    pltpu.sync_copy(mb, acc.at[idxd], add=True)

                nb = (nfull - 1 - s) // NTILES + 1

                def base_of(k):
                    return (s + k * NTILES) * K

                start(base_of(0), 0)

                def kstep(k, _, start=start, finish=finish):
                    for p in range(2):
                        @pl.when(k % 2 == p)
                        def _(p=p):
                            @pl.when(k + 1 < nb)
                            def _():
                                start(base_of(k + 1), 1 - p)
                            finish(p)
                    return 0
                lax.fori_loop(0, nb, kstep, 0)

        plsc.subcore_barrier()
        for gi, (nd, _) in enumerate(groups):
            acc = accs[gi]
            oa, ob = outs[2 * gi], outs[2 * gi + 1]

            def dump(r0, nr, acc=acc, oa=oa, ob=ob):
                @pl.when(c == 0)
                def _():
                    pltpu.sync_copy(acc.at[pl.ds(r0, nr)], oa.at[pl.ds(r0, nr)])

                @pl.when(c == 1)
                def _():
                    pltpu.sync_copy(acc.at[pl.ds(r0, nr)], ob.at[pl.ds(r0, nr)])
            _per_tile_rows(s, nd, dump)

    mesh = plsc.VectorSubcoreMesh(core_axis_name='c', subcore_axis_name='s')
    flat_in = [a for _, rels in groups for r in rels for a in r]
    outs = pl.kernel(body, out_type=out_type, mesh=mesh,
                     compiler_params=pltpu.CompilerParams(
                         use_tc_tiling_on_sc=False),
                     scratch_types=scratch)(*flat_in)
    return [tuple(outs[2 * g:2 * g + 2]) for g in range(len(groups))]


def _tc_pool(y, batch, bsum, block_r=512):
    n = y.shape[0]
    n_pad = pl.cdiv(n, block_r) * block_r
    y = jnp.pad(y, ((0, n_pad - n), (0, 0)))
    batch3 = jnp.pad(batch, (0, n_pad - n), constant_values=B).reshape(
        n_pad // block_r, 1, block_r)

    def body(y_ref, b_ref, bs_ref, s_ref, c_ref):
        i = pl.program_id(0)

        @pl.when(i == 0)
        def _():
            s_ref[...] = jnp.zeros_like(s_ref)
            c_ref[...] = jnp.zeros_like(c_ref)

        yb = y_ref[...] + bs_ref[...]
        ye = jnp.where(yb > 0, yb, jnp.exp(jnp.minimum(yb, 0.0)) - 1.0)
        ids = b_ref[0, 0, :]
        oh = (lax.broadcasted_iota(jnp.int32, (B, block_r), 0)
              == ids[None, :]).astype(jnp.float32)
        s_ref[...] += jnp.dot(oh, ye, preferred_element_type=jnp.float32)
        c_ref[...] = c_ref[...] + jnp.sum(oh, axis=1, keepdims=True)

    return pl.pallas_call(
        body,
        grid=(n_pad // block_r,),
        in_specs=[pl.BlockSpec((block_r, C), lambda i: (i, 0)),
                  pl.BlockSpec((1, 1, block_r), lambda i: (i, 0, 0)),
                  pl.BlockSpec((1, C), lambda i: (0, 0))],
        out_specs=[pl.BlockSpec((B, C), lambda i: (0, 0)),
                   pl.BlockSpec((B, C), lambda i: (0, 0))],
        out_shape=[jax.ShapeDtypeStruct((B, C), jnp.float32),
                   jax.ShapeDtypeStruct((B, C), jnp.float32)],
    )(y, batch3, bsum)


def _tc_head(pooled, lin_w, lin_b):
    def body(sc, cc, sv, cv, sk, ck, w_ref, b_ref, o_ref):
        z = jnp.concatenate(
            [sc[...] / jnp.maximum(cc[...], 1.0),
             sv[...] / jnp.maximum(cv[...], 1.0),
             sk[...] / jnp.maximum(ck[...], 1.0)], axis=1)
        logits = jnp.dot(z, w_ref[...], preferred_element_type=jnp.float32)
        logits = logits + b_ref[...]
        m = jnp.max(logits, axis=1, keepdims=True)
        e = jnp.exp(logits - m)
        o_ref[...] = (logits - m) - jnp.log(jnp.sum(e, axis=1, keepdims=True))

    args = [a for sc_cc in pooled for a in sc_cc] + [lin_w, lin_b.reshape(1, OUT)]
    return pl.pallas_call(
        body,
        out_shape=jax.ShapeDtypeStruct((B, OUT), jnp.float32),
    )(*args)


def kernel(x_control, x_variable, x_constant, params, edge_index_cc,
           edge_index_call, edge_index_cv, edge_index_vc, edge_index_kc,
           edge_index_ck, batch_control, batch_variable, batch_constant):
    eis = {'cc': edge_index_cc, 'call': edge_index_call, 'cv': edge_index_cv,
           'vc': edge_index_vc, 'kc': edge_index_kc, 'ck': edge_index_ck}
    batches = {'control': batch_control, 'variable': batch_variable,
               'constant': batch_constant}
    xd = {'control': x_control, 'variable': x_variable, 'constant': x_constant}
    src_t = {name: s for (s, _, name) in RELS}
    dst_t = {name: d for (_, d, name) in RELS}

    bsum = None
    for li, d_in in (('layer0', 128), ('layer1', C)):
        lp = params[li]
        wcat, widths = _build_wcat(lp, d_in)
        hsA, hsB, aS, aD = {}, {}, {}, {}
        bs_t = {t: (None if bsum is None else bsum[t]) for t in NTYPES}
        for t in NTYPES:
            aouts = _tc_matmul(xd[t], wcat[t][0], widths[t][0], bsum=bs_t[t])
            src_rels = [name for (s, _, name) in RELS if s == t]
            dst_rels = [name for (_, dt, name) in RELS if dt == t]
            for i, name in enumerate(src_rels):
                aS[name] = aouts[i]
            for i, name in enumerate(dst_rels):
                aD[name] = aouts[len(src_rels) + i]

        ex, dnm = {}, {}
        for pair in (['cc', 'vc'], ['call', 'kc'], ['cv', 'ck']):
            res = _sc_pass1([(eis[n][0], eis[n][1], aS[n], aD[n])
                             for n in pair])
            for n, (e_, d_) in zip(pair, res):
                ex[n], dnm[n] = e_, d_

        for t in NTYPES:
            houts = _tc_matmul(xd[t], wcat[t][1], widths[t][1], bsum=bs_t[t])
            src_rels = [name for (s, _, name) in RELS if s == t]
            for i, name in enumerate(src_rels):
                hsA[name], hsB[name] = houts[2 * i], houts[2 * i + 1]

        def rel_args(n):
            return (eis[n][0], eis[n][1], hsA[n], hsB[n], ex[n], dnm[n])

        (res_c,) = _sc_pass2([(xd['control'].shape[0],
                               [rel_args(n) for n in ('cc', 'call', 'vc', 'kc')])],
                             K=40, zr=64)
        res_v, res_k = _sc_pass2([
            (xd['variable'].shape[0], [rel_args('cv')]),
            (xd['constant'].shape[0], [rel_args('ck')])])

        nxt, bsum_n = {}, {}
        for t, (oa, ob) in zip(NTYPES, (res_c, res_v, res_k)):
            nxt[t] = jnp.concatenate([oa, ob], axis=1)
            bs = sum(lp[n]['bias'] for n in eis if dst_t[n] == t)
            bsum_n[t] = bs.reshape(1, C)
        xd, bsum = nxt, bsum_n

    pooled = [_tc_pool(xd[t], batches[t], bsum[t]) for t in NTYPES]
    return _tc_head(pooled, params['lin_w'], params['lin_b'])


# speedup vs baseline: 1.0617x; 1.0045x over previous
import functools

import jax
import jax.numpy as jnp
from jax import lax
from jax.experimental import pallas as pl
from jax.experimental.pallas import tpu as pltpu
from jax.experimental.pallas import tpu_sc as plsc

H = 8
C = 64
HALF = C // 2
AW = 16
B = 64
OUT = 10
RELS = [('control', 'control', 'cc'), ('control', 'control', 'call'),
        ('control', 'variable', 'cv'), ('variable', 'control', 'vc'),
        ('constant', 'control', 'kc'), ('control', 'constant', 'ck')]
NTYPES = ('control', 'variable', 'constant')
NSC = 2
NTILES = 16
ZR = 128


def _tc_matmul(x, w, widths, bsum=None, block_r=512):
    n, d = x.shape
    tot = w.shape[1]
    nin = 2 + (1 if bsum is not None else 0)

    def body(*refs):
        xb = refs[0][...]
        if bsum is not None:
            xb = xb + refs[2][...]
            xb = jnp.where(xb > 0, xb, jnp.exp(jnp.minimum(xb, 0.0)) - 1.0)
        acc = jnp.dot(xb, refs[1][...], preferred_element_type=jnp.float32)
        off = 0
        for o_ref in refs[nin:]:
            wd = o_ref.shape[1]
            o_ref[...] = acc[:, off:off + wd]
            off += wd

    in_specs = [pl.BlockSpec((block_r, d), lambda i: (i, 0)),
                pl.BlockSpec((d, tot), lambda i: (0, 0))]
    args = [x, w]
    if bsum is not None:
        in_specs.append(pl.BlockSpec((1, d), lambda i: (0, 0)))
        args.append(bsum)
    return pl.pallas_call(
        body,
        grid=(pl.cdiv(n, block_r),),
        in_specs=in_specs,
        out_specs=[pl.BlockSpec((block_r, wd), lambda i: (i, 0)) for wd in widths],
        out_shape=[jax.ShapeDtypeStruct((n, wd), jnp.float32) for wd in widths],
    )(*args)


def _build_wcat(lp, d_in):
    perm_a = [h * C + c for h in range(H) for c in range(HALF)]
    perm_b = [h * C + HALF + c for h in range(H) for c in range(HALF)]
    wcat, widths = {}, {}
    for t in NTYPES:
        src_rels = [name for (s, _, name) in RELS if s == t]
        dst_rels = [name for (_, dt, name) in RELS if dt == t]
        acols, awd = [], []
        for name in src_rels:
            p = lp[name]
            ws2 = jnp.einsum('dhc,hc->dh', p['w_src'].reshape(d_in, H, C),
                             p['att_src'][0])
            acols.append(jnp.pad(ws2, ((0, 0), (0, AW - H)))); awd.append(AW)
        for name in dst_rels:
            p = lp[name]
            wd2 = jnp.einsum('dhc,hc->dh', p['w_dst'].reshape(d_in, H, C),
                             p['att_dst'][0])
            acols.append(jnp.pad(wd2, ((0, 0), (0, AW - H)))); awd.append(AW)
        hcols, hwd = [], []
        for name in src_rels:
            w = lp[name]['w_src']
            hcols.append(w[:, perm_a]); hwd.append(H * HALF)
            hcols.append(w[:, perm_b]); hwd.append(H * HALF)
        wcat[t] = (jnp.concatenate(acols, axis=1),
                   jnp.concatenate(hcols, axis=1))
        widths[t] = (awd, hwd)
    return wcat, widths


def _tile_rows(nd):
    r8 = -(-(nd // NTILES) // 8) * 8
    last = nd - (NTILES - 1) * r8
    assert last >= 0 and last % 8 == 0
    return r8, last


def _zero_rows(acc, row0, nrows, zbuf):
    zr = zbuf.shape[0]
    nz, rem = nrows // zr, nrows % zr

    def zk(k, _):
        pltpu.sync_copy(zbuf, acc.at[pl.ds(row0 + k * zr, zr)])
        return 0
    lax.fori_loop(0, nz, zk, 0)
    if rem:
        pltpu.sync_copy(zbuf.at[pl.ds(0, rem)],
                        acc.at[pl.ds(row0 + nz * zr, rem)])


def _per_tile_rows(s, acc_nd, fn):
    r8, last = _tile_rows(acc_nd)

    @pl.when(s < NTILES - 1)
    def _():
        fn(s * r8, r8)

    @pl.when(s == NTILES - 1)
    def _():
        fn((NTILES - 1) * r8, last)


def _fill_zeros(zbuf):
    z = jnp.zeros((16,), jnp.float32)

    def zrow(i, _):
        for w in range(zbuf.shape[1] // 16):
            zbuf[i, pl.ds(w * 16, 16)] = z
        return 0
    lax.fori_loop(0, zbuf.shape[0], zrow, 0)


def _sc_pass1(rels):
    nrel = len(rels)
    nr = nrel // NSC
    assert nr * NSC == nrel
    E = rels[0][0].shape[0]
    K = 80
    nfull = E // K
    assert nfull * K == E and K % 8 == 0
    nds = [r[3].shape[0] for r in rels]
    acc_nds = [max(nds[j], nds[nr + j]) for j in range(nr)]

    out_type = []
    for nd in nds:
        out_type += [jax.ShapeDtypeStruct((E, AW), jnp.float32),
                     jax.ShapeDtypeStruct((nd, AW), jnp.float32)]
    p1set = [pltpu.VMEM((K,), jnp.int32), pltpu.VMEM((K,), jnp.int32),
             pltpu.VMEM((K, AW), jnp.float32), pltpu.VMEM((K, AW), jnp.float32)]
    scratch = (p1set + p1set
               + [pltpu.VMEM((K, AW), jnp.float32),
                  pltpu.VMEM((ZR, AW), jnp.float32),
                  pltpu.SemaphoreType.DMA, pltpu.SemaphoreType.DMA])
    scratch += [pltpu.VMEM_SHARED((nd, AW), jnp.float32) for nd in acc_nds]

    def body(*refs):
        ins = refs[:4 * nrel]
        outs = refs[4 * nrel:4 * nrel + 2 * nrel]
        scr = refs[4 * nrel + 2 * nrel:]
        bufs = (scr[0:4], scr[4:8])
        exb, zbuf = scr[8:10]
        sems = (scr[10], scr[11])
        accs = scr[12:]

        c = lax.axis_index('c')
        s = lax.axis_index('s')
        _fill_zeros(zbuf)

        def process(r, acc):
            src_h, dst_h, as_h, ad_h = ins[4 * r:4 * r + 4]
            ex_h = outs[2 * r]

            def start(base, p):
                bi_s, bi_d, ba_s, ba_d = bufs[p]
                sem = sems[p]
                i1 = pltpu.async_copy(src_h.at[pl.ds(base, K)], bi_s, sem)
                i2 = pltpu.async_copy(dst_h.at[pl.ds(base, K)], bi_d, sem)
                i1.wait(); i2.wait()
                pltpu.async_copy(as_h.at[bi_s], ba_s, sem)
                pltpu.async_copy(ad_h.at[bi_d], ba_d, sem)

            def finish(base, p):
                bi_s, bi_d, ba_s, ba_d = bufs[p]
                sem = sems[p]
                pltpu.make_async_copy(as_h.at[bi_s], ba_s, sem).wait()
                pltpu.make_async_copy(ad_h.at[bi_d], ba_d, sem).wait()

                def row(i, _):
                    v = ba_s[i, :] + ba_d[i, :]
                    v = jnp.where(v > 0, v, 0.2 * v)
                    exb[i, :] = jnp.exp(v)
                    return 0
                lax.fori_loop(0, K, row, 0)
                pltpu.sync_copy(exb, acc.at[bi_d], add=True)
                pltpu.sync_copy(exb, ex_h.at[pl.ds(base, K)])

            nb = (nfull - 1 - s) // NTILES + 1

            def base_of(k):
                return (s + k * NTILES) * K

            start(base_of(0), 0)

            def kstep(k, _):
                for p in range(2):
                    @pl.when(k % 2 == p)
                    def _(p=p):
                        @pl.when(k + 1 < nb)
                        def _():
                            start(base_of(k + 1), 1 - p)
                        finish(base_of(k), p)
                return 0
            lax.fori_loop(0, nb, kstep, 0)

        for sc in range(NSC):
            @pl.when(c == sc)
            def _(sc=sc):
                for j in range(nr):
                    r = sc * nr + j
                    _per_tile_rows(s, nds[r],
                                   lambda r0, nrw, j=j: _zero_rows(
                                       accs[j], r0, nrw, zbuf))
        plsc.subcore_barrier()

        for sc in range(NSC):
            @pl.when(c == sc)
            def _(sc=sc):
                for j in range(nr):
                    process(sc * nr + j, accs[j])
        plsc.subcore_barrier()

        for sc in range(NSC):
            @pl.when(c == sc)
            def _(sc=sc):
                for j in range(nr):
                    r = sc * nr + j
                    dn = outs[2 * r + 1]

                    def dump(r0, nrw, j=j, dn=dn):
                        pltpu.sync_copy(accs[j].at[pl.ds(r0, nrw)],
                                        dn.at[pl.ds(r0, nrw)])
                    _per_tile_rows(s, nds[r], dump)

    mesh = plsc.VectorSubcoreMesh(core_axis_name='c', subcore_axis_name='s')
    flat_in = [a for r in rels for a in r]
    outs = pl.kernel(body, out_type=out_type, mesh=mesh,
                     compiler_params=pltpu.CompilerParams(
                         use_tc_tiling_on_sc=False),
                     scratch_types=scratch)(*flat_in)
    return [tuple(outs[2 * r:2 * r + 2]) for r in range(nrel)]


def _sc_pass2(groups, K=64, zr=ZR):
    E = groups[0][1][0][0].shape[0]
    nfull = E // K
    assert nfull * K == E

    out_type = []
    for nd, _ in groups:
        out_type += [jax.ShapeDtypeStruct((nd, HALF), jnp.float32),
                     jax.ShapeDtypeStruct((nd, HALF), jnp.float32)]
    bufset = [pltpu.VMEM((K,), jnp.int32), pltpu.VMEM((K,), jnp.int32),
              pltpu.VMEM((K, H * HALF), jnp.float32),
              pltpu.VMEM((K, AW), jnp.float32),
              pltpu.VMEM((K, AW), jnp.float32)]
    scratch = (bufset + bufset
               + [pltpu.VMEM((K, AW), jnp.float32),
                  pltpu.VMEM((K, HALF), jnp.float32),
                  pltpu.VMEM((zr, HALF), jnp.float32),
                  pltpu.SemaphoreType.DMA, pltpu.SemaphoreType.DMA])
    scratch += [pltpu.VMEM_SHARED((nd, HALF), jnp.float32) for nd, _ in groups]

    nin = sum(6 * len(rels) for _, rels in groups)

    def body(*refs):
        ins = refs[:nin]
        outs = refs[nin:nin + 2 * len(groups)]
        scr = refs[nin + 2 * len(groups):]
        bufs = (scr[0:5], scr[5:10])
        attb, mb, zbuf = scr[10:13]
        sems = (scr[13], scr[14])
        accs = scr[15:]

        c = lax.axis_index('c')
        s = lax.axis_index('s')
        _fill_zeros(zbuf)
        for acc in accs:
            _per_tile_rows(s, acc.shape[0],
                           lambda r0, nr, acc=acc: _zero_rows(acc, r0, nr, zbuf))
        plsc.subcore_barrier()

        off = 0
        for gi, (nd, rels) in enumerate(groups):
            acc = accs[gi]
            for _ in rels:
                src_h, dst_h, hsa_h, hsb_h, ex_h, da_h = ins[off:off + 6]
                off += 6

                def start(base, p, src_h=src_h, dst_h=dst_h, hsa_h=hsa_h,
                          hsb_h=hsb_h, ex_h=ex_h, da_h=da_h):
                    idxs, idxd, hsb, exb, dab = bufs[p]
                    sem = sems[p]
                    i1 = pltpu.async_copy(src_h.at[pl.ds(base, K)], idxs, sem)
                    i2 = pltpu.async_copy(dst_h.at[pl.ds(base, K)], idxd, sem)
                    i1.wait(); i2.wait()

                    @pl.when(c == 0)
                    def _():
                        pltpu.async_copy(hsa_h.at[idxs], hsb, sem)

                    @pl.when(c == 1)
                    def _():
                        pltpu.async_copy(hsb_h.at[idxs], hsb, sem)
                    pltpu.async_copy(da_h.at[idxd], dab, sem)
                    pltpu.async_copy(ex_h.at[pl.ds(base, K)], exb, sem)

                def finish(p, hsa_h=hsa_h, ex_h=ex_h, da_h=da_h, acc=acc):
                    idxs, idxd, hsb, exb, dab = bufs[p]
                    sem = sems[p]
                    pltpu.make_async_copy(hsa_h.at[idxs], hsb, sem).wait()
                    pltpu.make_async_copy(da_h.at[idxd], dab, sem).wait()
                    pltpu.make_async_copy(ex_h.at[pl.ds(0, K)], exb, sem).wait()

                    def att_row(i, _):
                        attb[i, :] = exb[i, :] / (
                            (dab[i, :] + 1e-16) * float(H))
                        return 0
                    lax.fori_loop(0, K, att_row, 0)

                    def msg_row(i, _):
                        m0 = jnp.zeros((16,), jnp.float32)
                        m1 = jnp.zeros((16,), jnp.float32)
                        av = attb[i, :]
                        for h in range(H):
                            a = av[h]
                            m0 = m0 + a * hsb[i, pl.ds(h * HALF, 16)]
                            m1 = m1 + a * hsb[i, pl.ds(h * HALF + 16, 16)]
                        mb[i, pl.ds(0, 16)] = m0
                        mb[i, pl.ds(16, 16)] = m1
                        return 0
                    lax.fori_loop(0, K, msg_row, 0)
                    pltpu.sync_copy(mb, acc.at[idxd], add=True)

                nb = (nfull - 1 - s) // NTILES + 1

                def base_of(k):
                    return (s + k * NTILES) * K

                start(base_of(0), 0)

                def kstep(k, _, start=start, finish=finish):
                    for p in range(2):
                        @pl.when(k % 2 == p)
                        def _(p=p):
                            @pl.when(k + 1 < nb)
                            def _():
                                start(base_of(k + 1), 1 - p)
                            finish(p)
                    return 0
                lax.fori_loop(0, nb, kstep, 0)

        plsc.subcore_barrier()
        for gi, (nd, _) in enumerate(groups):
            acc = accs[gi]
            oa, ob = outs[2 * gi], outs[2 * gi + 1]

            def dump(r0, nr, acc=acc, oa=oa, ob=ob):
                @pl.when(c == 0)
                def _():
                    pltpu.sync_copy(acc.at[pl.ds(r0, nr)], oa.at[pl.ds(r0, nr)])

                @pl.when(c == 1)
                def _():
                    pltpu.sync_copy(acc.at[pl.ds(r0, nr)], ob.at[pl.ds(r0, nr)])
            _per_tile_rows(s, nd, dump)

    mesh = plsc.VectorSubcoreMesh(core_axis_name='c', subcore_axis_name='s')
    flat_in = [a for _, rels in groups for r in rels for a in r]
    outs = pl.kernel(body, out_type=out_type, mesh=mesh,
                     compiler_params=pltpu.CompilerParams(
                         use_tc_tiling_on_sc=False),
                     scratch_types=scratch)(*flat_in)
    return [tuple(outs[2 * g:2 * g + 2]) for g in range(len(groups))]


def _tc_pool(y, batch, bsum, block_r=512):
    n = y.shape[0]
    n_pad = pl.cdiv(n, block_r) * block_r
    y = jnp.pad(y, ((0, n_pad - n), (0, 0)))
    batch3 = jnp.pad(batch, (0, n_pad - n), constant_values=B).reshape(
        n_pad // block_r, 1, block_r)

    def body(y_ref, b_ref, bs_ref, s_ref, c_ref):
        i = pl.program_id(0)

        @pl.when(i == 0)
        def _():
            s_ref[...] = jnp.zeros_like(s_ref)
            c_ref[...] = jnp.zeros_like(c_ref)

        yb = y_ref[...] + bs_ref[...]
        ye = jnp.where(yb > 0, yb, jnp.exp(jnp.minimum(yb, 0.0)) - 1.0)
        ids = b_ref[0, 0, :]
        oh = (lax.broadcasted_iota(jnp.int32, (B, block_r), 0)
              == ids[None, :]).astype(jnp.float32)
        s_ref[...] += jnp.dot(oh, ye, preferred_element_type=jnp.float32)
        c_ref[...] = c_ref[...] + jnp.sum(oh, axis=1, keepdims=True)

    return pl.pallas_call(
        body,
        grid=(n_pad // block_r,),
        in_specs=[pl.BlockSpec((block_r, C), lambda i: (i, 0)),
                  pl.BlockSpec((1, 1, block_r), lambda i: (i, 0, 0)),
                  pl.BlockSpec((1, C), lambda i: (0, 0))],
        out_specs=[pl.BlockSpec((B, C), lambda i: (0, 0)),
                   pl.BlockSpec((B, C), lambda i: (0, 0))],
        out_shape=[jax.ShapeDtypeStruct((B, C), jnp.float32),
                   jax.ShapeDtypeStruct((B, C), jnp.float32)],
    )(y, batch3, bsum)


def _tc_head(pooled, lin_w, lin_b):
    def body(sc, cc, sv, cv, sk, ck, w_ref, b_ref, o_ref):
        z = jnp.concatenate(
            [sc[...] / jnp.maximum(cc[...], 1.0),
             sv[...] / jnp.maximum(cv[...], 1.0),
             sk[...] / jnp.maximum(ck[...], 1.0)], axis=1)
        logits = jnp.dot(z, w_ref[...], preferred_element_type=jnp.float32)
        logits = logits + b_ref[...]
        m = jnp.max(logits, axis=1, keepdims=True)
        e = jnp.exp(logits - m)
        o_ref[...] = (logits - m) - jnp.log(jnp.sum(e, axis=1, keepdims=True))

    args = [a for sc_cc in pooled for a in sc_cc] + [lin_w, lin_b.reshape(1, OUT)]
    return pl.pallas_call(
        body,
        out_shape=jax.ShapeDtypeStruct((B, OUT), jnp.float32),
    )(*args)


def kernel(x_control, x_variable, x_constant, params, edge_index_cc,
           edge_index_call, edge_index_cv, edge_index_vc, edge_index_kc,
           edge_index_ck, batch_control, batch_variable, batch_constant):
    eis = {'cc': edge_index_cc, 'call': edge_index_call, 'cv': edge_index_cv,
           'vc': edge_index_vc, 'kc': edge_index_kc, 'ck': edge_index_ck}
    batches = {'control': batch_control, 'variable': batch_variable,
               'constant': batch_constant}
    xd = {'control': x_control, 'variable': x_variable, 'constant': x_constant}
    src_t = {name: s for (s, _, name) in RELS}
    dst_t = {name: d for (_, d, name) in RELS}

    bsum = None
    for li, d_in in (('layer0', 128), ('layer1', C)):
        lp = params[li]
        wcat, widths = _build_wcat(lp, d_in)
        hsA, hsB, aS, aD = {}, {}, {}, {}
        bs_t = {t: (None if bsum is None else bsum[t]) for t in NTYPES}
        for t in NTYPES:
            aouts = _tc_matmul(xd[t], wcat[t][0], widths[t][0], bsum=bs_t[t])
            src_rels = [name for (s, _, name) in RELS if s == t]
            dst_rels = [name for (_, dt, name) in RELS if dt == t]
            for i, name in enumerate(src_rels):
                aS[name] = aouts[i]
            for i, name in enumerate(dst_rels):
                aD[name] = aouts[len(src_rels) + i]

        ex, dnm = {}, {}
        for pair in (['cc', 'vc'], ['call', 'kc'], ['cv', 'ck']):
            res = _sc_pass1([(eis[n][0], eis[n][1], aS[n], aD[n])
                             for n in pair])
            for n, (e_, d_) in zip(pair, res):
                ex[n], dnm[n] = e_, d_

        for t in NTYPES:
            houts = _tc_matmul(xd[t], wcat[t][1], widths[t][1], bsum=bs_t[t])
            src_rels = [name for (s, _, name) in RELS if s == t]
            for i, name in enumerate(src_rels):
                hsA[name], hsB[name] = houts[2 * i], houts[2 * i + 1]

        def rel_args(n):
            return (eis[n][0], eis[n][1], hsA[n], hsB[n], ex[n], dnm[n])

        (res_c,) = _sc_pass2([(xd['control'].shape[0],
                               [rel_args(n) for n in ('cc', 'call', 'vc', 'kc')])],
                             K=40, zr=64)
        res_v, res_k = _sc_pass2([
            (xd['variable'].shape[0], [rel_args('cv')]),
            (xd['constant'].shape[0], [rel_args('ck')])], K=80)

        nxt, bsum_n = {}, {}
        for t, (oa, ob) in zip(NTYPES, (res_c, res_v, res_k)):
            nxt[t] = jnp.concatenate([oa, ob], axis=1)
            bs = sum(lp[n]['bias'] for n in eis if dst_t[n] == t)
            bsum_n[t] = bs.reshape(1, C)
        xd, bsum = nxt, bsum_n

    pooled = [_tc_pool(xd[t], batches[t], bsum[t]) for t in NTYPES]
    return _tc_head(pooled, params['lin_w'], params['lin_b'])
